# baseline (speedup 1.0000x reference)
import functools

import jax
import jax.numpy as jnp
from jax import lax
from jax.experimental import pallas as pl
from jax.experimental.pallas import tpu as pltpu
from jax.experimental.pallas import tpu_sc as plsc

_N = 10000
_E = 320000
_F = 128
_H = 128
_STEPS = 3

_NC = 2
_NS = 16
_NW = _NC * _NS
_EPW = _E // _NW
_CHUNK = 128
_NCH = 79
_EPW_PAD = _NCH * _CHUNK
_AGG_ROWS = 10240
_DUMMY_ROW = _N
_BLK = 1000


def _segsum_body(m_hbm, idx_hbm, out_hbm, rows_v, idx_v, agg_sh, gsem):
    c = lax.axis_index("c")
    s = lax.axis_index("s")
    wid = c * _NS + s

    def _zfill(r, carry):
        for k in range(_H // 16):
            rows_v[r, pl.ds(k * 16, 16)] = jnp.zeros((16,), jnp.float32)
        return carry
    lax.fori_loop(0, _CHUNK, _zfill, 0)
    for kk in range(_AGG_ROWS // _NS // _CHUNK):
        pltpu.sync_copy(rows_v,
                        agg_sh.at[pl.ds(s * (_AGG_ROWS // _NS) + kk * _CHUNK,
                                        _CHUNK)])

    pltpu.sync_copy(idx_hbm.at[wid], idx_v)
    plsc.subcore_barrier()

    off = wid * _NCH // _NW

    def _edge(i, carry):
        j = i + off
        j = jnp.where(j >= _NCH, j - _NCH, j)
        pltpu.async_copy(m_hbm.at[idx_v.at[j, 0]], rows_v, gsem).wait()
        pltpu.sync_copy(rows_v, agg_sh.at[idx_v.at[j, 1]], add=True)
        return carry
    lax.fori_loop(0, _NCH, _edge, 0)
    plsc.subcore_barrier()

    rows_out = _AGG_ROWS // _NS
    pltpu.sync_copy(agg_sh.at[pl.ds(s * rows_out, rows_out)],
                    out_hbm.at[c, pl.ds(s * rows_out, rows_out)])


@jax.jit
def _segsum(m, idx_p):
    mesh = plsc.VectorSubcoreMesh(core_axis_name="c", subcore_axis_name="s")
    f = pl.kernel(
        _segsum_body,
        out_type=jax.ShapeDtypeStruct((_NC, _AGG_ROWS, _H), jnp.float32),
        mesh=mesh,
        scratch_types=[
            pltpu.VMEM((_CHUNK, _H), jnp.float32),
            pltpu.VMEM((_NCH, 2, _CHUNK), jnp.int32),
            pltpu.VMEM_SHARED((_AGG_ROWS, _H), jnp.float32),
            pltpu.SemaphoreType.DMA,
        ],
    )
    return f(m, idx_p)


_NBKT = 64
_BKTW = 157


def _bucket(sv):
    return lax.shift_right_logical(sv * 213723, 25)


def _binsort_body(idx_hbm, out_hbm, idx_v, out_v, hist_v, pos_v):
    c = lax.axis_index("c")
    s = lax.axis_index("s")
    wid = c * _NS + s
    pltpu.sync_copy(idx_hbm.at[wid], idx_v)
    lane = lax.iota(jnp.int32, 16)
    zeros16 = jnp.zeros((16,), jnp.int32)

    for g in range(_NBKT):
        plsc.store_scatter(hist_v, [g * 16 + lane], zeros16)

    def _p1(j, carry):
        for k in range(_CHUNK // 16):
            sv = idx_v[j, 0, pl.ds(k * 16, 16)]
            hidx = _bucket(sv) * 16 + lane
            cur = plsc.load_gather(hist_v, [hidx])
            plsc.store_scatter(hist_v, [hidx], cur + 1)
        return carry
    lax.fori_loop(0, _NCH, _p1, 0)

    carry = jnp.zeros((), jnp.int32)
    for g in range(_NBKT):
        v = plsc.load_gather(hist_v, [g * 16 + lane])
        inc = plsc.cumsum(v)
        plsc.store_scatter(pos_v, [g * 16 + lane], inc - v + carry)
        carry = carry + jnp.sum(v)

    def _p2(j, carry):
        for k in range(_CHUNK // 16):
            sv = idx_v[j, 0, pl.ds(k * 16, 16)]
            dv = idx_v[j, 1, pl.ds(k * 16, 16)]
            hidx = _bucket(sv) * 16 + lane
            p = plsc.load_gather(pos_v, [hidx])
            plsc.store_scatter(pos_v, [hidx], p + 1)
            so = p + lax.shift_right_logical(p, 7) * 128
            plsc.store_scatter(out_v, [so], sv)
            plsc.store_scatter(out_v, [so + 128], dv)
        return carry
    lax.fori_loop(0, _NCH, _p2, 0)
    pltpu.sync_copy(out_v, out_hbm.at[wid])


@jax.jit
def _binsort(idx_p):
    mesh = plsc.VectorSubcoreMesh(core_axis_name="c", subcore_axis_name="s")
    f = pl.kernel(
        _binsort_body,
        out_type=jax.ShapeDtypeStruct((_NW, _NCH * 2 * _CHUNK), jnp.int32),
        mesh=mesh,
        compiler_params=pltpu.CompilerParams(needs_layout_passes=False),
        scratch_types=[
            pltpu.VMEM((_NCH, 2, _CHUNK), jnp.int32),
            pltpu.VMEM((_NCH * 2 * _CHUNK,), jnp.int32),
            pltpu.VMEM((16 * _NBKT,), jnp.int32),
            pltpu.VMEM((16 * _NBKT,), jnp.int32),
        ],
    )
    return f(idx_p)


def _proj_body(x_ref, wi_ref, wm_ref, h_ref, m_ref):
    h = jnp.dot(x_ref[...], wi_ref[...], preferred_element_type=jnp.float32)
    h_ref[...] = h
    m_ref[...] = jnp.dot(h, wm_ref[...], preferred_element_type=jnp.float32)


@jax.jit
def _proj(x, wi_t, wm0):
    return pl.pallas_call(
        _proj_body,
        grid=(_N // _BLK,),
        in_specs=[
            pl.BlockSpec((_BLK, _F), lambda i: (i, 0)),
            pl.BlockSpec((_F, _H), lambda i: (0, 0)),
            pl.BlockSpec((_H, _H), lambda i: (0, 0)),
        ],
        out_specs=[
            pl.BlockSpec((_BLK, _H), lambda i: (i, 0)),
            pl.BlockSpec((_BLK, _H), lambda i: (i, 0)),
        ],
        out_shape=[
            jax.ShapeDtypeStruct((_N, _H), jnp.float32),
            jax.ShapeDtypeStruct((_N, _H), jnp.float32),
        ],
    )(x, wi_t, wm0)


def _gru_math(a0, a1, h, wih_t, whh_t, bih, bhh):
    agg = a0[0] + a1[0]
    gi = jnp.dot(agg, wih_t, preferred_element_type=jnp.float32) + bih
    gh = jnp.dot(h, whh_t, preferred_element_type=jnp.float32) + bhh
    r = jax.nn.sigmoid(gi[:, :_H] + gh[:, :_H])
    z = jax.nn.sigmoid(gi[:, _H:2 * _H] + gh[:, _H:2 * _H])
    n = jnp.tanh(gi[:, 2 * _H:] + r * gh[:, 2 * _H:])
    return (1.0 - z) * n + z * h


def _gru_next_body(a0_ref, a1_ref, h_ref, wih_ref, whh_ref, bih_ref, bhh_ref,
                   wm_ref, hout_ref, mout_ref):
    hn = _gru_math(a0_ref[...], a1_ref[...], h_ref[...], wih_ref[...],
                   whh_ref[...], bih_ref[...], bhh_ref[...])
    hout_ref[...] = hn
    mout_ref[...] = jnp.dot(hn, wm_ref[...], preferred_element_type=jnp.float32)


def _gru_last_body(a0_ref, a1_ref, h_ref, wih_ref, whh_ref, bih_ref, bhh_ref,
                   hout_ref):
    hout_ref[...] = _gru_math(a0_ref[...], a1_ref[...], h_ref[...],
                              wih_ref[...], whh_ref[...], bih_ref[...],
                              bhh_ref[...])


_GRU_IN_SPECS = [
    pl.BlockSpec((1, _BLK, _H), lambda i: (0, i, 0)),
    pl.BlockSpec((1, _BLK, _H), lambda i: (1, i, 0)),
    pl.BlockSpec((_BLK, _H), lambda i: (i, 0)),
    pl.BlockSpec((_H, 3 * _H), lambda i: (0, 0)),
    pl.BlockSpec((_H, 3 * _H), lambda i: (0, 0)),
    pl.BlockSpec((1, 3 * _H), lambda i: (0, 0)),
    pl.BlockSpec((1, 3 * _H), lambda i: (0, 0)),
]


@jax.jit
def _gru_next(a0, a1, h, wih_t, whh_t, bih, bhh, wm):
    return pl.pallas_call(
        _gru_next_body,
        grid=(_N // _BLK,),
        in_specs=_GRU_IN_SPECS + [pl.BlockSpec((_H, _H), lambda i: (0, 0))],
        out_specs=[
            pl.BlockSpec((_BLK, _H), lambda i: (i, 0)),
            pl.BlockSpec((_BLK, _H), lambda i: (i, 0)),
        ],
        out_shape=[
            jax.ShapeDtypeStruct((_N, _H), jnp.float32),
            jax.ShapeDtypeStruct((_N, _H), jnp.float32),
        ],
    )(a0, a1, h, wih_t, whh_t, bih, bhh, wm)


@jax.jit
def _gru_last(a0, a1, h, wih_t, whh_t, bih, bhh):
    return pl.pallas_call(
        _gru_last_body,
        grid=(_N // _BLK,),
        in_specs=_GRU_IN_SPECS,
        out_specs=pl.BlockSpec((_BLK, _H), lambda i: (i, 0)),
        out_shape=jax.ShapeDtypeStruct((_N, _H), jnp.float32),
    )(a0, a1, h, wih_t, whh_t, bih, bhh)


def _final_body(h_ref, wp_ref, bp_ref, out_ref, acc_ref):
    i = pl.program_id(0)

    @pl.when(i == 0)
    def _():
        acc_ref[...] = jnp.zeros_like(acc_ref)

    hb = h_ref[...]
    leak = jnp.where(hb > 0, hb, 0.01 * hb)
    acc_ref[...] += jnp.sum(leak, axis=0, keepdims=True)

    @pl.when(i == pl.num_programs(0) - 1)
    def _():
        g = acc_ref[...] / _N
        out_ref[...] = (jnp.sum(g * wp_ref[...], axis=1, keepdims=True)
                        + bp_ref[...])


@jax.jit
def _final(h, wp, bp):
    return pl.pallas_call(
        _final_body,
        grid=(_N // _BLK,),
        in_specs=[
            pl.BlockSpec((_BLK, _H), lambda i: (i, 0)),
            pl.BlockSpec((1, _H), lambda i: (0, 0)),
            pl.BlockSpec((1, 1), lambda i: (0, 0)),
        ],
        out_specs=pl.BlockSpec((1, 1), lambda i: (0, 0)),
        out_shape=jax.ShapeDtypeStruct((1, 1), jnp.float32),
        scratch_shapes=[pltpu.VMEM((1, _H), jnp.float32)],
    )(h, wp, bp)


def kernel(x, edge_index, W_input, W_mpnn, W_ih, W_hh, b_ih, b_hh,
           W_pred, b_pred):
    wi_t = W_input.T
    wih_t = W_ih.T
    whh_t = W_hh.T
    bih = b_ih.reshape(1, 3 * _H)
    bhh = b_hh.reshape(1, 3 * _H)

    ei = edge_index.astype(jnp.int32)
    pad = _EPW_PAD - _EPW
    src_p = jnp.pad(ei[0].reshape(_NW, _EPW), ((0, 0), (0, pad)),
                    constant_values=0).reshape(_NW, _NCH, _CHUNK)
    dst_p = jnp.pad(ei[1].reshape(_NW, _EPW), ((0, 0), (0, pad)),
                    constant_values=_DUMMY_ROW).reshape(_NW, _NCH, _CHUNK)
    idx_p = jnp.stack([src_p, dst_p], axis=2)
    idx_s = _binsort(idx_p).reshape(_NW, _NCH, 2, _CHUNK)

    h, m = _proj(x, wi_t, W_mpnn[0])
    for t in range(_STEPS):
        aggp = _segsum(m, idx_s)
        if t < _STEPS - 1:
            h, m = _gru_next(aggp, aggp, h, wih_t, whh_t, bih, bhh,
                             W_mpnn[t + 1])
        else:
            h = _gru_last(aggp, aggp, h, wih_t, whh_t, bih, bhh)
    out = _final(h, W_pred, b_pred.reshape(1, 1))
    return out.reshape(1)


# speedup vs baseline: 1.0133x; 1.0133x over previous
import functools

import jax
import jax.numpy as jnp
from jax import lax
from jax.experimental import pallas as pl
from jax.experimental.pallas import tpu as pltpu
from jax.experimental.pallas import tpu_sc as plsc

_N = 10000
_E = 320000
_F = 128
_H = 128
_STEPS = 3

_NC = 2
_NS = 16
_NW = _NC * _NS
_EPW = _E // _NW
_CHUNK = 128
_NCH = 79
_EPW_PAD = _NCH * _CHUNK
_AGG_ROWS = 10240
_DUMMY_ROW = _N
_BLK = 1000


def _segsum_body(m_hbm, idx_hbm, out_hbm, rows_v, idx_v, agg_sh, gsem):
    c = lax.axis_index("c")
    s = lax.axis_index("s")
    wid = c * _NS + s

    def _zfill(r, carry):
        for k in range(_H // 16):
            rows_v[r, pl.ds(k * 16, 16)] = jnp.zeros((16,), jnp.float32)
        return carry
    lax.fori_loop(0, _CHUNK, _zfill, 0)
    for kk in range(_AGG_ROWS // _NS // _CHUNK):
        pltpu.sync_copy(rows_v,
                        agg_sh.at[pl.ds(s * (_AGG_ROWS // _NS) + kk * _CHUNK,
                                        _CHUNK)])

    pltpu.sync_copy(idx_hbm.at[wid], idx_v)
    plsc.subcore_barrier()

    off = wid * _NCH // _NW

    def _edge(i, carry):
        j = i + off
        j = jnp.where(j >= _NCH, j - _NCH, j)
        pltpu.async_copy(m_hbm.at[idx_v.at[j, 0]], rows_v, gsem).wait()
        pltpu.sync_copy(rows_v, agg_sh.at[idx_v.at[j, 1]], add=True)
        return carry
    lax.fori_loop(0, _NCH, _edge, 0)
    plsc.subcore_barrier()

    rows_out = _AGG_ROWS // _NS
    pltpu.sync_copy(agg_sh.at[pl.ds(s * rows_out, rows_out)],
                    out_hbm.at[c, pl.ds(s * rows_out, rows_out)])


@jax.jit
def _segsum(m, idx_p):
    mesh = plsc.VectorSubcoreMesh(core_axis_name="c", subcore_axis_name="s")
    f = pl.kernel(
        _segsum_body,
        out_type=jax.ShapeDtypeStruct((_NC, _AGG_ROWS, _H), jnp.float32),
        mesh=mesh,
        scratch_types=[
            pltpu.VMEM((_CHUNK, _H), jnp.float32),
            pltpu.VMEM((_NCH, 2, _CHUNK), jnp.int32),
            pltpu.VMEM_SHARED((_AGG_ROWS, _H), jnp.float32),
            pltpu.SemaphoreType.DMA,
        ],
    )
    return f(m, idx_p)


_NBKT = 32
_BKTW = 313


def _bucket(sv):
    return lax.shift_right_logical(sv * 107203, 25)


def _binsort_body(idx_hbm, out_hbm, idx_v, out_v, hist_v, pos_v):
    c = lax.axis_index("c")
    s = lax.axis_index("s")
    wid = c * _NS + s
    pltpu.sync_copy(idx_hbm.at[wid], idx_v)
    lane = lax.iota(jnp.int32, 16)
    zeros16 = jnp.zeros((16,), jnp.int32)

    for g in range(_NBKT):
        plsc.store_scatter(hist_v, [g * 16 + lane], zeros16)

    def _p1(j, carry):
        for k in range(_CHUNK // 16):
            sv = idx_v[j, 0, pl.ds(k * 16, 16)]
            hidx = _bucket(sv) * 16 + lane
            cur = plsc.load_gather(hist_v, [hidx])
            plsc.store_scatter(hist_v, [hidx], cur + 1)
        return carry
    lax.fori_loop(0, _NCH, _p1, 0)

    carry = jnp.zeros((), jnp.int32)
    for g in range(_NBKT):
        v = plsc.load_gather(hist_v, [g * 16 + lane])
        inc = plsc.cumsum(v)
        plsc.store_scatter(pos_v, [g * 16 + lane], inc - v + carry)
        carry = carry + jnp.sum(v)

    def _p2(j, carry):
        for k in range(_CHUNK // 16):
            sv = idx_v[j, 0, pl.ds(k * 16, 16)]
            dv = idx_v[j, 1, pl.ds(k * 16, 16)]
            hidx = _bucket(sv) * 16 + lane
            p = plsc.load_gather(pos_v, [hidx])
            plsc.store_scatter(pos_v, [hidx], p + 1)
            so = p + lax.shift_right_logical(p, 7) * 128
            plsc.store_scatter(out_v, [so], sv)
            plsc.store_scatter(out_v, [so + 128], dv)
        return carry
    lax.fori_loop(0, _NCH, _p2, 0)
    pltpu.sync_copy(out_v, out_hbm.at[wid])


@jax.jit
def _binsort(idx_p):
    mesh = plsc.VectorSubcoreMesh(core_axis_name="c", subcore_axis_name="s")
    f = pl.kernel(
        _binsort_body,
        out_type=jax.ShapeDtypeStruct((_NW, _NCH * 2 * _CHUNK), jnp.int32),
        mesh=mesh,
        compiler_params=pltpu.CompilerParams(needs_layout_passes=False),
        scratch_types=[
            pltpu.VMEM((_NCH, 2, _CHUNK), jnp.int32),
            pltpu.VMEM((_NCH * 2 * _CHUNK,), jnp.int32),
            pltpu.VMEM((16 * _NBKT,), jnp.int32),
            pltpu.VMEM((16 * _NBKT,), jnp.int32),
        ],
    )
    return f(idx_p)


def _proj_body(x_ref, wi_ref, wm_ref, h_ref, m_ref):
    h = jnp.dot(x_ref[...], wi_ref[...], preferred_element_type=jnp.float32)
    h_ref[...] = h
    m_ref[...] = jnp.dot(h, wm_ref[...], preferred_element_type=jnp.float32)


@jax.jit
def _proj(x, wi_t, wm0):
    return pl.pallas_call(
        _proj_body,
        grid=(_N // _BLK,),
        in_specs=[
            pl.BlockSpec((_BLK, _F), lambda i: (i, 0)),
            pl.BlockSpec((_F, _H), lambda i: (0, 0)),
            pl.BlockSpec((_H, _H), lambda i: (0, 0)),
        ],
        out_specs=[
            pl.BlockSpec((_BLK, _H), lambda i: (i, 0)),
            pl.BlockSpec((_BLK, _H), lambda i: (i, 0)),
        ],
        out_shape=[
            jax.ShapeDtypeStruct((_N, _H), jnp.float32),
            jax.ShapeDtypeStruct((_N, _H), jnp.float32),
        ],
    )(x, wi_t, wm0)


def _gru_math(a0, a1, h, wih_t, whh_t, bih, bhh):
    agg = a0[0] + a1[0]
    gi = jnp.dot(agg, wih_t, preferred_element_type=jnp.float32) + bih
    gh = jnp.dot(h, whh_t, preferred_element_type=jnp.float32) + bhh
    r = jax.nn.sigmoid(gi[:, :_H] + gh[:, :_H])
    z = jax.nn.sigmoid(gi[:, _H:2 * _H] + gh[:, _H:2 * _H])
    n = jnp.tanh(gi[:, 2 * _H:] + r * gh[:, 2 * _H:])
    return (1.0 - z) * n + z * h


def _gru_next_body(a0_ref, a1_ref, h_ref, wih_ref, whh_ref, bih_ref, bhh_ref,
                   wm_ref, hout_ref, mout_ref):
    hn = _gru_math(a0_ref[...], a1_ref[...], h_ref[...], wih_ref[...],
                   whh_ref[...], bih_ref[...], bhh_ref[...])
    hout_ref[...] = hn
    mout_ref[...] = jnp.dot(hn, wm_ref[...], preferred_element_type=jnp.float32)


def _gru_last_body(a0_ref, a1_ref, h_ref, wih_ref, whh_ref, bih_ref, bhh_ref,
                   wp_ref, bp_ref, out_ref, acc_ref):
    i = pl.program_id(0)

    @pl.when(i == 0)
    def _():
        acc_ref[...] = jnp.zeros_like(acc_ref)

    hn = _gru_math(a0_ref[...], a1_ref[...], h_ref[...], wih_ref[...],
                   whh_ref[...], bih_ref[...], bhh_ref[...])
    leak = jnp.where(hn > 0, hn, 0.01 * hn)
    acc_ref[...] += jnp.sum(leak, axis=0, keepdims=True)

    @pl.when(i == pl.num_programs(0) - 1)
    def _():
        g = acc_ref[...] / _N
        out_ref[...] = (jnp.sum(g * wp_ref[...], axis=1, keepdims=True)
                        + bp_ref[...])


_GRU_IN_SPECS = [
    pl.BlockSpec((1, _BLK, _H), lambda i: (0, i, 0)),
    pl.BlockSpec((1, _BLK, _H), lambda i: (1, i, 0)),
    pl.BlockSpec((_BLK, _H), lambda i: (i, 0)),
    pl.BlockSpec((_H, 3 * _H), lambda i: (0, 0)),
    pl.BlockSpec((_H, 3 * _H), lambda i: (0, 0)),
    pl.BlockSpec((1, 3 * _H), lambda i: (0, 0)),
    pl.BlockSpec((1, 3 * _H), lambda i: (0, 0)),
]


@jax.jit
def _gru_next(a0, a1, h, wih_t, whh_t, bih, bhh, wm):
    return pl.pallas_call(
        _gru_next_body,
        grid=(_N // _BLK,),
        in_specs=_GRU_IN_SPECS + [pl.BlockSpec((_H, _H), lambda i: (0, 0))],
        out_specs=[
            pl.BlockSpec((_BLK, _H), lambda i: (i, 0)),
            pl.BlockSpec((_BLK, _H), lambda i: (i, 0)),
        ],
        out_shape=[
            jax.ShapeDtypeStruct((_N, _H), jnp.float32),
            jax.ShapeDtypeStruct((_N, _H), jnp.float32),
        ],
    )(a0, a1, h, wih_t, whh_t, bih, bhh, wm)


@jax.jit
def _gru_last(a0, a1, h, wih_t, whh_t, bih, bhh, wp, bp):
    return pl.pallas_call(
        _gru_last_body,
        grid=(_N // _BLK,),
        in_specs=_GRU_IN_SPECS + [
            pl.BlockSpec((1, _H), lambda i: (0, 0)),
            pl.BlockSpec((1, 1), lambda i: (0, 0)),
        ],
        out_specs=pl.BlockSpec((1, 1), lambda i: (0, 0)),
        out_shape=jax.ShapeDtypeStruct((1, 1), jnp.float32),
        scratch_shapes=[pltpu.VMEM((1, _H), jnp.float32)],
    )(a0, a1, h, wih_t, whh_t, bih, bhh, wp, bp)


def kernel(x, edge_index, W_input, W_mpnn, W_ih, W_hh, b_ih, b_hh,
           W_pred, b_pred):
    wi_t = W_input.T
    wih_t = W_ih.T
    whh_t = W_hh.T
    bih = b_ih.reshape(1, 3 * _H)
    bhh = b_hh.reshape(1, 3 * _H)

    ei = edge_index.astype(jnp.int32)
    pad = _EPW_PAD - _EPW
    src_p = jnp.pad(ei[0].reshape(_NW, _EPW), ((0, 0), (0, pad)),
                    constant_values=0).reshape(_NW, _NCH, _CHUNK)
    dst_p = jnp.pad(ei[1].reshape(_NW, _EPW), ((0, 0), (0, pad)),
                    constant_values=_DUMMY_ROW).reshape(_NW, _NCH, _CHUNK)
    idx_p = jnp.stack([src_p, dst_p], axis=2)
    idx_s = _binsort(idx_p).reshape(_NW, _NCH, 2, _CHUNK)

    h, m = _proj(x, wi_t, W_mpnn[0])
    for t in range(_STEPS):
        aggp = _segsum(m, idx_s)
        if t < _STEPS - 1:
            h, m = _gru_next(aggp, aggp, h, wih_t, whh_t, bih, bhh,
                             W_mpnn[t + 1])
        else:
            out = _gru_last(aggp, aggp, h, wih_t, whh_t, bih, bhh,
                            W_pred, b_pred.reshape(1, 1))
    return out.reshape(1)
